# Initial kernel scaffold; baseline (speedup 1.0000x reference)
import jax
import jax.numpy as jnp
from jax.experimental import pallas as pl


def kernel(x, table, conv_w, conv_b, gamma, beta):
    raise NotImplementedError("write your pallas kernel here")


# speedup vs baseline: 5.2699x; 5.2699x over previous
import functools

import jax
import jax.numpy as jnp
from jax import lax
from jax.experimental import pallas as pl
from jax.experimental.pallas import tpu as pltpu
from jax.experimental.pallas import tpu_sc as plsc

_V = 15000
_D = 64
_B, _S, _L = 256, 12, 600
_N = _B * _S * _L
_NC, _NS, _LANES = 2, 16, 16
_NW = _NC * _NS
_PER_W = _N // _NW
_VPAD = 15360
_GRP = 128
_K = 5
_CHUNK = _K * _GRP
_STEPS = _PER_W // _CHUNK

_mesh = plsc.VectorSubcoreMesh(core_axis_name="c", subcore_axis_name="s")


@functools.partial(
    pl.kernel,
    mesh=_mesh,
    out_type=jax.ShapeDtypeStruct((_NW, _VPAD), jnp.float32),
    scratch_types=[
        pltpu.VMEM((_PER_W,), jnp.int32),
        pltpu.VMEM((_VPAD,), jnp.float32),
    ],
    compiler_params=pltpu.CompilerParams(needs_layout_passes=False),
)
def _hist(x_hbm, out_hbm, idx_v, hist_v):
    wid = lax.axis_index("s") * _NC + lax.axis_index("c")

    zeros = jnp.zeros((_LANES,), jnp.float32)

    def zero_body(i, carry):
        hist_v[pl.ds(i * _LANES, _LANES)] = zeros
        return carry

    lax.fori_loop(0, _VPAD // _LANES, zero_body, 0)

    pltpu.sync_copy(x_hbm.at[pl.ds(wid * _PER_W, _PER_W)], idx_v)

    ones = jnp.ones((_LANES,), jnp.float32)

    def body(i, carry):
        idx16 = idx_v[pl.ds(i * _LANES, _LANES)]
        plsc.addupdate_scatter(hist_v, [idx16], ones)
        return carry

    lax.fori_loop(0, _PER_W // _LANES, body, 0)

    pltpu.sync_copy(hist_v, out_hbm.at[wid])


def _stats_body(table_ref, wt_ref, b_ref, gamma_ref, beta_ref, counts_ref,
                t3_ref):
    t2 = jnp.dot(table_ref[...], wt_ref[...],
                 preferred_element_type=jnp.float32) + b_ref[...]
    c = jnp.sum(counts_ref[...], axis=0, keepdims=True)
    n = jnp.float32(_N)
    mean = jnp.dot(c, t2, preferred_element_type=jnp.float32) / n
    sq = jnp.dot(c, t2 * t2, preferred_element_type=jnp.float32) / n
    var = sq - mean * mean
    scale = gamma_ref[...] * lax.rsqrt(var + 1e-5)
    shift = beta_ref[...] - mean * scale
    t3_ref[...] = jnp.clip(t2 * scale + shift, 0.0, 6.0)


@jax.jit
def _stats(table_pad, conv_wt, conv_b, gamma, beta, counts):
    return pl.pallas_call(
        _stats_body,
        out_shape=jax.ShapeDtypeStruct((_VPAD, _D), jnp.float32),
    )(table_pad, conv_wt, conv_b, gamma, beta, counts)


@functools.partial(
    pl.kernel,
    mesh=_mesh,
    out_type=jax.ShapeDtypeStruct((_N, _D), jnp.float32),
    scratch_types=[
        pltpu.VMEM_SHARED((_VPAD, _D), jnp.float32),
        pltpu.VMEM((_CHUNK,), jnp.int32),
        pltpu.VMEM((_CHUNK, _D), jnp.float32),
        pltpu.SemaphoreType.DMA,
    ],
    compiler_params=pltpu.CompilerParams(use_tc_tiling_on_sc=False),
)
def _gather(x_hbm, t3_hbm, out_hbm, t3_sh, idx_v, rows_v, sem):
    sid = lax.axis_index("s")
    wid = sid * _NC + lax.axis_index("c")
    row0 = wid * _PER_W

    @pl.when(sid == 0)
    def _stage():
        pltpu.sync_copy(t3_hbm, t3_sh)

    plsc.subcore_barrier()

    def step(st, carry):
        base = row0 + st * _CHUNK
        pltpu.sync_copy(x_hbm.at[pl.ds(base, _CHUNK)], idx_v)
        copies = [
            pltpu.async_copy(t3_sh.at[idx_v.at[pl.ds(j * _GRP, _GRP)]],
                             rows_v.at[pl.ds(j * _GRP, _GRP)], sem)
            for j in range(_K)
        ]
        for c in copies:
            c.wait()
        pltpu.sync_copy(rows_v, out_hbm.at[pl.ds(base, _CHUNK)])
        return carry

    lax.fori_loop(0, _STEPS, step, 0)


@jax.jit
def kernel(x, table, conv_w, conv_b, gamma, beta):
    xf = x.reshape(_N).astype(jnp.int32)
    counts = _hist(xf)
    table_pad = jnp.concatenate(
        [table, jnp.zeros((_VPAD - _V, _D), jnp.float32)], axis=0)
    t3 = _stats(table_pad, conv_w.T, conv_b.reshape(1, _D),
                gamma.reshape(1, _D), beta.reshape(1, _D), counts)
    out = _gather(xf, t3)
    return out.reshape(_B, _S, _L, _D)


# speedup vs baseline: 5.2725x; 1.0005x over previous
import functools

import jax
import jax.numpy as jnp
from jax import lax
from jax.experimental import pallas as pl
from jax.experimental.pallas import tpu as pltpu
from jax.experimental.pallas import tpu_sc as plsc

_V = 15000
_D = 64
_B, _S, _L = 256, 12, 600
_N = _B * _S * _L
_NC, _NS, _LANES = 2, 16, 16
_NW = _NC * _NS
_PER_W = _N // _NW
_VPAD = 15360
_GRP = 128
_PANEL = _S * _L // _S
_PANELS_PER_W = _PER_W // _PANEL
_GRP_SIZES = (128, 128, 128, 128, 88)

_mesh = plsc.VectorSubcoreMesh(core_axis_name="c", subcore_axis_name="s")


@functools.partial(
    pl.kernel,
    mesh=_mesh,
    out_type=jax.ShapeDtypeStruct((_NW, _VPAD), jnp.float32),
    scratch_types=[
        pltpu.VMEM((_PER_W,), jnp.int32),
        pltpu.VMEM((_VPAD,), jnp.float32),
    ],
    compiler_params=pltpu.CompilerParams(needs_layout_passes=False),
)
def _hist(x_hbm, out_hbm, idx_v, hist_v):
    wid = lax.axis_index("s") * _NC + lax.axis_index("c")

    zeros = jnp.zeros((_LANES,), jnp.float32)

    def zero_body(i, carry):
        hist_v[pl.ds(i * _LANES, _LANES)] = zeros
        return carry

    lax.fori_loop(0, _VPAD // _LANES, zero_body, 0)

    pltpu.sync_copy(x_hbm.at[pl.ds(wid * _PER_W, _PER_W)], idx_v)

    ones = jnp.ones((_LANES,), jnp.float32)

    def body(i, carry):
        idx16 = idx_v[pl.ds(i * _LANES, _LANES)]
        plsc.addupdate_scatter(hist_v, [idx16], ones)
        return carry

    lax.fori_loop(0, _PER_W // _LANES, body, 0)

    pltpu.sync_copy(hist_v, out_hbm.at[wid])


def _stats_body(table_ref, wt_ref, b_ref, gamma_ref, beta_ref, counts_ref,
                t3_ref):
    t2 = jnp.dot(table_ref[...], wt_ref[...],
                 preferred_element_type=jnp.float32) + b_ref[...]
    c = jnp.sum(counts_ref[...], axis=0, keepdims=True)
    n = jnp.float32(_N)
    mean = jnp.dot(c, t2, preferred_element_type=jnp.float32) / n
    sq = jnp.dot(c, t2 * t2, preferred_element_type=jnp.float32) / n
    var = sq - mean * mean
    scale = gamma_ref[...] * lax.rsqrt(var + 1e-5)
    shift = beta_ref[...] - mean * scale
    t3_ref[...] = jnp.clip(t2 * scale + shift, 0.0, 6.0)


@jax.jit
def _stats(table_pad, conv_wt, conv_b, gamma, beta, counts):
    return pl.pallas_call(
        _stats_body,
        out_shape=jax.ShapeDtypeStruct((_VPAD, _D), jnp.float32),
    )(table_pad, conv_wt, conv_b, gamma, beta, counts)


@functools.partial(
    pl.kernel,
    mesh=_mesh,
    out_type=jax.ShapeDtypeStruct((_B, _S, _L, _D), jnp.float32),
    scratch_types=[
        pltpu.VMEM_SHARED((_VPAD, _D), jnp.float32),
        pltpu.VMEM((_PANEL,), jnp.int32),
        pltpu.VMEM((_PANEL, _D), jnp.float32),
        pltpu.SemaphoreType.DMA,
    ],
    compiler_params=pltpu.CompilerParams(use_tc_tiling_on_sc=False),
)
def _gather(x_hbm, t3_hbm, out_hbm, t3_sh, idx_v, rows_v, sem):
    sid = lax.axis_index("s")
    wid = sid * _NC + lax.axis_index("c")
    panel0 = wid * _PANELS_PER_W

    @pl.when(sid == 0)
    def _stage():
        pltpu.sync_copy(t3_hbm, t3_sh)

    plsc.subcore_barrier()

    def step(p, carry):
        panel = panel0 + p
        bi = panel // _S
        si = panel % _S
        pltpu.sync_copy(x_hbm.at[pl.ds(panel * _PANEL, _PANEL)], idx_v)
        copies = []
        off = 0
        for sz in _GRP_SIZES:
            copies.append(
                pltpu.async_copy(t3_sh.at[idx_v.at[pl.ds(off, sz)]],
                                 rows_v.at[pl.ds(off, sz)], sem))
            off += sz
        for c in copies:
            c.wait()
        pltpu.sync_copy(rows_v, out_hbm.at[bi, si])
        return carry

    lax.fori_loop(0, _PANELS_PER_W, step, 0)


@jax.jit
def kernel(x, table, conv_w, conv_b, gamma, beta):
    xf = x.reshape(_N).astype(jnp.int32)
    counts = _hist(xf)
    table_pad = jnp.concatenate(
        [table, jnp.zeros((_VPAD - _V, _D), jnp.float32)], axis=0)
    t3 = _stats(table_pad, conv_w.T, conv_b.reshape(1, _D),
                gamma.reshape(1, _D), beta.reshape(1, _D), counts)
    return _gather(xf, t3)


# speedup vs baseline: 5.9219x; 1.1232x over previous
import functools

import jax
import jax.numpy as jnp
from jax import lax
from jax.experimental import pallas as pl
from jax.experimental.pallas import tpu as pltpu
from jax.experimental.pallas import tpu_sc as plsc

_V = 15000
_D = 64
_B, _S, _L = 256, 12, 600
_N = _B * _S * _L
_NC, _NS, _LANES = 2, 16, 16
_NW = _NC * _NS
_PER_W = _N // _NW
_VPAD = 15360
_CHUNK = 480
_GRPS = ((0, 128), (128, 128), (256, 128), (384, 96))
_STEPS = _PER_W // _CHUNK
_ITERS = _STEPS // 2

_mesh = plsc.VectorSubcoreMesh(core_axis_name="c", subcore_axis_name="s")


@functools.partial(
    pl.kernel,
    mesh=_mesh,
    out_type=jax.ShapeDtypeStruct((_NW, _VPAD), jnp.float32),
    scratch_types=[
        pltpu.VMEM((_PER_W,), jnp.int32),
        pltpu.VMEM((_VPAD,), jnp.float32),
    ],
    compiler_params=pltpu.CompilerParams(needs_layout_passes=False),
)
def _hist(x_hbm, out_hbm, idx_v, hist_v):
    wid = lax.axis_index("s") * _NC + lax.axis_index("c")

    zeros = jnp.zeros((_LANES,), jnp.float32)

    def zero_body(i, carry):
        hist_v[pl.ds(i * _LANES, _LANES)] = zeros
        return carry

    lax.fori_loop(0, _VPAD // _LANES, zero_body, 0)

    pltpu.sync_copy(x_hbm.at[pl.ds(wid * _PER_W, _PER_W)], idx_v)

    ones = jnp.ones((_LANES,), jnp.float32)

    def body(i, carry):
        for u in range(8):
            idx16 = idx_v[pl.ds((i * 8 + u) * _LANES, _LANES)]
            plsc.addupdate_scatter(hist_v, [idx16], ones)
        return carry

    lax.fori_loop(0, _PER_W // (_LANES * 8), body, 0)

    pltpu.sync_copy(hist_v, out_hbm.at[wid])


def _stats_body(table_ref, wt_ref, b_ref, gamma_ref, beta_ref, counts_ref,
                t3_ref):
    t2 = jnp.dot(table_ref[...], wt_ref[...],
                 preferred_element_type=jnp.float32) + b_ref[...]
    c = jnp.sum(counts_ref[...], axis=0, keepdims=True)
    n = jnp.float32(_N)
    mean = jnp.dot(c, t2, preferred_element_type=jnp.float32) / n
    sq = jnp.dot(c, t2 * t2, preferred_element_type=jnp.float32) / n
    var = sq - mean * mean
    scale = gamma_ref[...] * lax.rsqrt(var + 1e-5)
    shift = beta_ref[...] - mean * scale
    t3_ref[...] = jnp.clip(t2 * scale + shift, 0.0, 6.0)


@jax.jit
def _stats(table_pad, conv_wt, conv_b, gamma, beta, counts):
    return pl.pallas_call(
        _stats_body,
        out_shape=jax.ShapeDtypeStruct((_VPAD, _D), jnp.float32),
    )(table_pad, conv_wt, conv_b, gamma, beta, counts)


@functools.partial(
    pl.kernel,
    mesh=_mesh,
    out_type=jax.ShapeDtypeStruct((_N, _D), jnp.float32),
    scratch_types=[
        pltpu.VMEM_SHARED((_VPAD, _D), jnp.float32),
        pltpu.VMEM((_CHUNK,), jnp.int32),
        pltpu.VMEM((_CHUNK,), jnp.int32),
        pltpu.VMEM((_CHUNK, _D), jnp.float32),
        pltpu.VMEM((_CHUNK, _D), jnp.float32),
        pltpu.SemaphoreType.DMA,
        pltpu.SemaphoreType.DMA,
        pltpu.SemaphoreType.DMA,
    ],
    compiler_params=pltpu.CompilerParams(use_tc_tiling_on_sc=False),
)
def _gather(x_hbm, t3_hbm, out_hbm, t3_sh, idx_a, idx_b, rows_a, rows_b,
            sem_g, sem_w, sem_i):
    sid = lax.axis_index("s")
    wid = sid * _NC + lax.axis_index("c")
    row0 = wid * _PER_W

    @pl.when(sid == 0)
    def _stage():
        pltpu.sync_copy(t3_hbm, t3_sh)

    plsc.subcore_barrier()

    def _fire(idx_v, rows_v):
        return [
            pltpu.async_copy(t3_sh.at[idx_v.at[pl.ds(off, sz)]],
                             rows_v.at[pl.ds(off, sz)], sem_g)
            for off, sz in _GRPS
        ]

    def _drain_gathers(rows_v):
        for off, sz in _GRPS:
            pltpu.make_async_copy(t3_hbm.at[pl.ds(0, sz)],
                                  rows_v.at[pl.ds(off, sz)], sem_g).wait()

    pltpu.sync_copy(x_hbm.at[pl.ds(row0, _CHUNK)], idx_a)
    _fire(idx_a, rows_a)
    pltpu.async_copy(x_hbm.at[pl.ds(row0 + _CHUNK, _CHUNK)], idx_b, sem_i)

    def body(i, carry):
        r_a = row0 + (2 * i) * _CHUNK
        _drain_gathers(rows_a)

        @pl.when(i > 0)
        def _wait_write_b_prev():
            pltpu.make_async_copy(rows_b, out_hbm.at[pl.ds(0, _CHUNK)],
                                  sem_w).wait()

        write_a = pltpu.async_copy(rows_a, out_hbm.at[pl.ds(r_a, _CHUNK)],
                                   sem_w)
        pltpu.make_async_copy(x_hbm.at[pl.ds(0, _CHUNK)], idx_b, sem_i).wait()
        copies_b = _fire(idx_b, rows_b)

        @pl.when(i < _ITERS - 1)
        def _prefetch_idx_a():
            pltpu.async_copy(x_hbm.at[pl.ds(r_a + 2 * _CHUNK, _CHUNK)],
                             idx_a, sem_i)

        for c in copies_b:
            c.wait()
        write_a.wait()
        pltpu.async_copy(rows_b, out_hbm.at[pl.ds(r_a + _CHUNK, _CHUNK)],
                         sem_w)

        @pl.when(i < _ITERS - 1)
        def _next_a():
            pltpu.make_async_copy(x_hbm.at[pl.ds(0, _CHUNK)], idx_a,
                                  sem_i).wait()
            _fire(idx_a, rows_a)
            pltpu.async_copy(x_hbm.at[pl.ds(r_a + 3 * _CHUNK, _CHUNK)],
                             idx_b, sem_i)

        return carry

    lax.fori_loop(0, _ITERS, body, 0)
    pltpu.make_async_copy(rows_b, out_hbm.at[pl.ds(0, _CHUNK)], sem_w).wait()


@jax.jit
def kernel(x, table, conv_w, conv_b, gamma, beta):
    xf = x.reshape(_N).astype(jnp.int32)
    counts = _hist(xf)
    table_pad = jnp.concatenate(
        [table, jnp.zeros((_VPAD - _V, _D), jnp.float32)], axis=0)
    t3 = _stats(table_pad, conv_w.T, conv_b.reshape(1, _D),
                gamma.reshape(1, _D), beta.reshape(1, _D), counts)
    return _gather(xf, t3).reshape(_B, _S, _L, _D)


# speedup vs baseline: 5.9913x; 1.0117x over previous
import functools

import jax
import jax.numpy as jnp
from jax import lax
from jax.experimental import pallas as pl
from jax.experimental.pallas import tpu as pltpu
from jax.experimental.pallas import tpu_sc as plsc

_V = 15000
_D = 64
_B, _S, _L = 256, 12, 600
_N = _B * _S * _L
_NC, _NS, _LANES = 2, 16, 16
_NW = _NC * _NS
_PER_W = _N // _NW
_VPAD = 15360
_CHUNK = 480
_GRPS = ((0, 128), (128, 128), (256, 128), (384, 96))
_STEPS = _PER_W // _CHUNK
_ITERS = _STEPS // 2

_mesh = plsc.VectorSubcoreMesh(core_axis_name="c", subcore_axis_name="s")


@functools.partial(
    pl.kernel,
    mesh=_mesh,
    out_type=jax.ShapeDtypeStruct((_NW, _VPAD), jnp.float32),
    scratch_types=[
        pltpu.VMEM((_PER_W,), jnp.int32),
        pltpu.VMEM((_VPAD,), jnp.float32),
    ],
    compiler_params=pltpu.CompilerParams(needs_layout_passes=False),
)
def _hist(x_hbm, out_hbm, idx_v, hist_v):
    wid = lax.axis_index("s") * _NC + lax.axis_index("c")

    zeros = jnp.zeros((_LANES,), jnp.float32)

    def zero_body(i, carry):
        hist_v[pl.ds(i * _LANES, _LANES)] = zeros
        return carry

    lax.fori_loop(0, _VPAD // _LANES, zero_body, 0)

    pltpu.sync_copy(x_hbm.at[pl.ds(wid * _PER_W, _PER_W)], idx_v)

    ones = jnp.ones((_LANES,), jnp.float32)

    def body(i, carry):
        idxs = [idx_v[pl.ds((i * 8 + u) * _LANES, _LANES)] for u in range(8)]
        for idx16 in idxs:
            plsc.addupdate_scatter(hist_v, [idx16], ones)
        return carry

    lax.fori_loop(0, _PER_W // (_LANES * 8), body, 0)

    pltpu.sync_copy(hist_v, out_hbm.at[wid])


def _stats_body(table_ref, wt_ref, b_ref, gamma_ref, beta_ref, counts_ref,
                t3_ref):
    t2 = jnp.dot(table_ref[...], wt_ref[...],
                 preferred_element_type=jnp.float32) + b_ref[...]
    c = jnp.sum(counts_ref[...], axis=0, keepdims=True)
    n = jnp.float32(_N)
    mean = jnp.dot(c, t2, preferred_element_type=jnp.float32) / n
    sq = jnp.dot(c, t2 * t2, preferred_element_type=jnp.float32) / n
    var = sq - mean * mean
    scale = gamma_ref[...] * lax.rsqrt(var + 1e-5)
    shift = beta_ref[...] - mean * scale
    t3_ref[...] = jnp.clip(t2 * scale + shift, 0.0, 6.0)


@jax.jit
def _stats(table_pad, conv_wt, conv_b, gamma, beta, counts):
    return pl.pallas_call(
        _stats_body,
        out_shape=jax.ShapeDtypeStruct((_VPAD, _D), jnp.float32),
    )(table_pad, conv_wt, conv_b, gamma, beta, counts)


@functools.partial(
    pl.kernel,
    mesh=_mesh,
    out_type=jax.ShapeDtypeStruct((_N, _D), jnp.float32),
    scratch_types=[
        pltpu.VMEM_SHARED((_VPAD, _D), jnp.float32),
        pltpu.VMEM((_CHUNK,), jnp.int32),
        pltpu.VMEM((_CHUNK,), jnp.int32),
        pltpu.VMEM((_CHUNK, _D), jnp.float32),
        pltpu.VMEM((_CHUNK, _D), jnp.float32),
        pltpu.SemaphoreType.DMA,
        pltpu.SemaphoreType.DMA,
        pltpu.SemaphoreType.DMA,
    ],
    compiler_params=pltpu.CompilerParams(use_tc_tiling_on_sc=False),
)
def _gather(x_hbm, t3_hbm, out_hbm, t3_sh, idx_a, idx_b, rows_a, rows_b,
            sem_g, sem_w, sem_i):
    sid = lax.axis_index("s")
    wid = sid * _NC + lax.axis_index("c")
    row0 = wid * _PER_W

    @pl.when(sid == 0)
    def _stage():
        pltpu.sync_copy(t3_hbm, t3_sh)

    plsc.subcore_barrier()

    def _fire(idx_v, rows_v):
        return [
            pltpu.async_copy(t3_sh.at[idx_v.at[pl.ds(off, sz)]],
                             rows_v.at[pl.ds(off, sz)], sem_g)
            for off, sz in _GRPS
        ]

    def _drain_gathers(rows_v):
        for off, sz in _GRPS:
            pltpu.make_async_copy(t3_hbm.at[pl.ds(0, sz)],
                                  rows_v.at[pl.ds(off, sz)], sem_g).wait()

    pltpu.sync_copy(x_hbm.at[pl.ds(row0, _CHUNK)], idx_a)
    _fire(idx_a, rows_a)
    pltpu.async_copy(x_hbm.at[pl.ds(row0 + _CHUNK, _CHUNK)], idx_b, sem_i)

    def body(i, carry):
        r_a = row0 + (2 * i) * _CHUNK
        _drain_gathers(rows_a)

        @pl.when(i > 0)
        def _wait_write_b_prev():
            pltpu.make_async_copy(rows_b, out_hbm.at[pl.ds(0, _CHUNK)],
                                  sem_w).wait()

        write_a = pltpu.async_copy(rows_a, out_hbm.at[pl.ds(r_a, _CHUNK)],
                                   sem_w)
        pltpu.make_async_copy(x_hbm.at[pl.ds(0, _CHUNK)], idx_b, sem_i).wait()
        copies_b = _fire(idx_b, rows_b)

        @pl.when(i < _ITERS - 1)
        def _prefetch_idx_a():
            pltpu.async_copy(x_hbm.at[pl.ds(r_a + 2 * _CHUNK, _CHUNK)],
                             idx_a, sem_i)

        for c in copies_b:
            c.wait()
        write_a.wait()
        pltpu.async_copy(rows_b, out_hbm.at[pl.ds(r_a + _CHUNK, _CHUNK)],
                         sem_w)

        @pl.when(i < _ITERS - 1)
        def _next_a():
            pltpu.make_async_copy(x_hbm.at[pl.ds(0, _CHUNK)], idx_a,
                                  sem_i).wait()
            _fire(idx_a, rows_a)
            pltpu.async_copy(x_hbm.at[pl.ds(r_a + 3 * _CHUNK, _CHUNK)],
                             idx_b, sem_i)

        return carry

    lax.fori_loop(0, _ITERS, body, 0)
    pltpu.make_async_copy(rows_b, out_hbm.at[pl.ds(0, _CHUNK)], sem_w).wait()


@jax.jit
def kernel(x, table, conv_w, conv_b, gamma, beta):
    xf = x.reshape(_N).astype(jnp.int32)
    counts = _hist(xf)
    table_pad = jnp.concatenate(
        [table, jnp.zeros((_VPAD - _V, _D), jnp.float32)], axis=0)
    t3 = _stats(table_pad, conv_w.T, conv_b.reshape(1, _D),
                gamma.reshape(1, _D), beta.reshape(1, _D), counts)
    return _gather(xf, t3).reshape(_B, _S, _L, _D)
